# baseline (speedup 1.0000x reference)
import functools

import jax
import jax.numpy as jnp
from jax import lax
from jax.experimental import pallas as pl
from jax.experimental.pallas import tpu as pltpu
from jax.experimental.pallas import tpu_sc as plsc

NUM_EMBEDDINGS = 1000000
DIM = 64
B = 16384
T = 50
BLK = 128
OPAD = 137
UNITS = (B // BLK) * T
NW = 32
UPW = UNITS // NW


def _make_gather():
    mesh = plsc.VectorSubcoreMesh(core_axis_name="c", subcore_axis_name="s")

    @functools.partial(
        pl.kernel,
        mesh=mesh,
        out_type=jax.ShapeDtypeStruct((T, 8, B // BLK, 8, BLK), jnp.float32),
        scratch_types=[
            [pltpu.VMEM((BLK,), jnp.int32)] * 2,
            [pltpu.VMEM((BLK, DIM), jnp.float32)] * 2,
            [pltpu.VMEM((8, 8, OPAD), jnp.float32)] * 2,
            [pltpu.SemaphoreType.DMA] * 2,
            [pltpu.SemaphoreType.DMA] * 2,
        ],
        compiler_params=pltpu.CompilerParams(
            use_tc_tiling_on_sc=False, needs_layout_passes=False
        ),
    )
    def gather_kernel(ids_hbm, w_hbm, out_hbm, idsb, g, ob, gsems, osems):
        wid = lax.axis_index("s") * 2 + lax.axis_index("c")
        u0 = wid * UPW
        iota = lax.iota(jnp.int32, 16)
        rows = [lax.add(iota, 16 * c0) for c0 in range(DIM // 16)]
        rows_i = [lax.shift_right_logical(r, 3) for r in rows]
        rows_ci = [lax.bitwise_and(r, 7) for r in rows]

        def unit_tj(i):
            u = u0 + i
            return u >> 7, u & 127

        def prep(p, i):
            t, j = unit_tj(i)
            pltpu.sync_copy(ids_hbm.at[t, pl.ds(j * BLK, BLK)], idsb[p])
            pltpu.async_copy(w_hbm.at[idsb[p]], g[p], gsems[p])

        def g_wait(p):
            pltpu.make_async_copy(w_hbm.at[idsb[p]], g[p], gsems[p]).wait()

        def o_src(p):
            return ob[p].at[:, :, pl.ds(0, BLK)]

        def o_start(p, i):
            t, j = unit_tj(i)
            pltpu.async_copy(o_src(p), out_hbm.at[t, :, j, :, :], osems[p])

        def o_wait(p, i):
            t, j = unit_tj(i)
            pltpu.make_async_copy(
                o_src(p), out_hbm.at[t, :, j, :, :], osems[p]
            ).wait()

        def select(p):
            one = lax.full((16,), 1, jnp.int32)

            def krow(j, col):
                for r in range(4):
                    k = 4 * j + r
                    for c0 in range(DIM // 16):
                        v = g[p][k, pl.ds(16 * c0, 16)]
                        plsc.store_scatter(
                            ob[p], [rows_i[c0], rows_ci[c0], col], v
                        )
                    col = lax.add(col, one)
                return col

            lax.fori_loop(0, BLK // 4, krow, lax.full((16,), 0, jnp.int32))

        def fin(p, i, guarded_owait):
            g_wait(p)
            if guarded_owait is None:
                o_wait(p, i - 2)
            else:
                @pl.when(guarded_owait)
                def _():
                    o_wait(p, i - 2)

            select(p)
            o_start(p, i)

        prep(0, 0)
        prep(1, 1)

        def body(j, carry):
            i = 2 * j
            fin(0, i, j > 0)
            prep(0, i + 2)
            fin(1, i + 1, j > 0)
            prep(1, i + 3)
            return carry

        lax.fori_loop(0, UPW // 2 - 1, body, 0)

        last = UPW - 2
        fin(0, last, None)
        fin(1, last + 1, None)
        o_wait(0, last)
        o_wait(1, last + 1)

    return gather_kernel


_gather = _make_gather()


@jax.jit
def kernel(input_ids, weight):
    ids_t = input_ids.T
    out5 = _gather(ids_t, weight)
    out = out5.transpose(2, 4, 0, 1, 3).reshape(B, T, DIM)
    return out


# speedup vs baseline: 1.1250x; 1.1250x over previous
import functools

import jax
import jax.numpy as jnp
from jax import lax
from jax.experimental import pallas as pl
from jax.experimental.pallas import tpu as pltpu
from jax.experimental.pallas import tpu_sc as plsc

NUM_EMBEDDINGS = 1000000
DIM = 64
B = 16384
T = 50
BLK = 128
OPAD = 136
UNITS = (B // BLK) * T
NW = 32
UPW = UNITS // NW


def _make_gather():
    mesh = plsc.VectorSubcoreMesh(core_axis_name="c", subcore_axis_name="s")

    @functools.partial(
        pl.kernel,
        mesh=mesh,
        out_type=jax.ShapeDtypeStruct((T, 8, B // BLK, 8, BLK), jnp.float32),
        scratch_types=[
            [pltpu.VMEM((BLK,), jnp.int32)] * 2,
            [pltpu.VMEM((BLK, DIM), jnp.float32)] * 2,
            [pltpu.VMEM((8, 8, OPAD), jnp.float32)] * 2,
            [pltpu.SemaphoreType.DMA] * 2,
            [pltpu.SemaphoreType.DMA] * 2,
        ],
        compiler_params=pltpu.CompilerParams(
            use_tc_tiling_on_sc=False, needs_layout_passes=False
        ),
    )
    def gather_kernel(ids_hbm, w_hbm, out_hbm, idsb, g, ob, gsems, osems):
        wid = lax.axis_index("s") * 2 + lax.axis_index("c")
        u0 = wid * UPW
        iota = lax.iota(jnp.int32, 16)
        rows = [lax.add(iota, 16 * c0) for c0 in range(DIM // 16)]
        rows_i = [lax.shift_right_logical(r, 3) for r in rows]
        rows_ci = [lax.bitwise_and(r, 7) for r in rows]

        def unit_tj(i):
            u = u0 + i
            return u >> 7, u & 127

        def prep(p, i):
            t, j = unit_tj(i)
            pltpu.sync_copy(ids_hbm.at[t, pl.ds(j * BLK, BLK)], idsb[p])
            pltpu.async_copy(w_hbm.at[idsb[p]], g[p], gsems[p])

        def g_wait(p):
            pltpu.make_async_copy(w_hbm.at[idsb[p]], g[p], gsems[p]).wait()

        def o_src(p):
            return ob[p].at[:, :, pl.ds(0, BLK)]

        def o_start(p, i):
            t, j = unit_tj(i)
            pltpu.async_copy(o_src(p), out_hbm.at[t, :, j, :, :], osems[p])

        def o_wait(p, i):
            t, j = unit_tj(i)
            pltpu.make_async_copy(
                o_src(p), out_hbm.at[t, :, j, :, :], osems[p]
            ).wait()

        def select(p):
            one = lax.full((16,), 1, jnp.int32)

            def krow(j, col):
                for r in range(4):
                    k = 4 * j + r
                    for c0 in range(DIM // 16):
                        v = g[p][k, pl.ds(16 * c0, 16)]
                        plsc.store_scatter(
                            ob[p], [rows_i[c0], rows_ci[c0], col], v
                        )
                    col = lax.add(col, one)
                return col

            lax.fori_loop(0, BLK // 4, krow, lax.full((16,), 0, jnp.int32))

        def fin(p, i, guarded_owait):
            g_wait(p)
            if guarded_owait is None:
                o_wait(p, i - 2)
            else:
                @pl.when(guarded_owait)
                def _():
                    o_wait(p, i - 2)

            select(p)
            o_start(p, i)

        prep(0, 0)
        prep(1, 1)

        def body(j, carry):
            i = 2 * j
            fin(0, i, j > 0)
            prep(0, i + 2)
            fin(1, i + 1, j > 0)
            prep(1, i + 3)
            return carry

        lax.fori_loop(0, UPW // 2 - 1, body, 0)

        last = UPW - 2
        fin(0, last, None)
        fin(1, last + 1, None)
        o_wait(0, last)
        o_wait(1, last + 1)

    return gather_kernel


_gather = _make_gather()


@jax.jit
def kernel(input_ids, weight):
    ids_t = input_ids.T
    out5 = _gather(ids_t, weight)
    out = out5.transpose(2, 4, 0, 1, 3).reshape(B, T, DIM)
    return out


# speedup vs baseline: 1.2674x; 1.1265x over previous
import functools

import jax
import jax.numpy as jnp
from jax import lax
from jax.experimental import pallas as pl
from jax.experimental.pallas import tpu as pltpu
from jax.experimental.pallas import tpu_sc as plsc

NUM_EMBEDDINGS = 1000000
DIM = 64
B = 16384
T = 50
BLK = 128
OPAD = 136
UNITS = (B // BLK) * T
NW = 32
UPW = UNITS // NW


def _make_gather():
    mesh = plsc.VectorSubcoreMesh(core_axis_name="c", subcore_axis_name="s")

    @functools.partial(
        pl.kernel,
        mesh=mesh,
        out_type=jax.ShapeDtypeStruct((T, 8, B // BLK, 8, BLK), jnp.float32),
        scratch_types=[
            pltpu.VMEM((UPW * BLK,), jnp.int32),
            [pltpu.VMEM((BLK, DIM), jnp.float32)] * 2,
            [pltpu.VMEM((8, 8, OPAD), jnp.float32)] * 2,
            [pltpu.SemaphoreType.DMA] * 2,
            [pltpu.SemaphoreType.DMA] * 2,
        ],
        compiler_params=pltpu.CompilerParams(
            use_tc_tiling_on_sc=False, needs_layout_passes=False
        ),
    )
    def gather_kernel(ids_hbm, w_hbm, out_hbm, idsall, g, ob, gsems, osems):
        wid = lax.axis_index("s") * 2 + lax.axis_index("c")
        u0 = wid * UPW
        pltpu.sync_copy(ids_hbm.at[pl.ds(u0 * BLK, UPW * BLK)], idsall)
        iota = lax.iota(jnp.int32, 16)
        rows = [lax.add(iota, 16 * c0) for c0 in range(DIM // 16)]
        rows_i = [lax.shift_right_logical(r, 3) for r in rows]
        rows_ci = [lax.bitwise_and(r, 7) for r in rows]

        def unit_tj(i):
            u = u0 + i
            return u >> 7, u & 127

        def prep(p, i):
            idx = idsall.at[pl.ds(i * BLK, BLK)]
            pltpu.async_copy(w_hbm.at[idx], g[p], gsems[p])

        def g_wait(p, i):
            idx = idsall.at[pl.ds(i * BLK, BLK)]
            pltpu.make_async_copy(w_hbm.at[idx], g[p], gsems[p]).wait()

        def o_src(p):
            return ob[p].at[:, :, pl.ds(0, BLK)]

        def o_start(p, i):
            t, j = unit_tj(i)
            pltpu.async_copy(o_src(p), out_hbm.at[t, :, j, :, :], osems[p])

        def o_wait(p, i):
            t, j = unit_tj(i)
            pltpu.make_async_copy(
                o_src(p), out_hbm.at[t, :, j, :, :], osems[p]
            ).wait()

        def select(p):
            one = lax.full((16,), 1, jnp.int32)

            def krow(j, col):
                for r in range(4):
                    k = 4 * j + r
                    for c0 in range(DIM // 16):
                        v = g[p][k, pl.ds(16 * c0, 16)]
                        plsc.store_scatter(
                            ob[p], [rows_i[c0], rows_ci[c0], col], v
                        )
                    col = lax.add(col, one)
                return col

            lax.fori_loop(0, BLK // 4, krow, lax.full((16,), 0, jnp.int32))

        def fin(p, i, guarded_owait):
            g_wait(p, i)
            if guarded_owait is None:
                o_wait(p, i - 2)
            else:
                @pl.when(guarded_owait)
                def _():
                    o_wait(p, i - 2)

            select(p)
            o_start(p, i)

        prep(0, 0)
        prep(1, 1)

        def body(j, carry):
            i = 2 * j
            fin(0, i, j > 0)
            prep(0, i + 2)
            fin(1, i + 1, j > 0)
            prep(1, i + 3)
            return carry

        lax.fori_loop(0, UPW // 2 - 1, body, 0)

        last = UPW - 2
        fin(0, last, None)
        fin(1, last + 1, None)
        o_wait(0, last)
        o_wait(1, last + 1)

    return gather_kernel


_gather = _make_gather()


@jax.jit
def kernel(input_ids, weight):
    ids_flat = input_ids.T.reshape(T * B)
    out5 = _gather(ids_flat, weight)
    out = out5.transpose(2, 4, 0, 1, 3).reshape(B, T, DIM)
    return out
